# Initial kernel scaffold; baseline (speedup 1.0000x reference)
import jax
import jax.numpy as jnp
from jax.experimental import pallas as pl


def kernel(edge_index, edge_weight, edge_type, embeddings, edge_type_emb):
    raise NotImplementedError("write your pallas kernel here")


# speedup vs baseline: 1.3697x; 1.3697x over previous
import functools

import jax
import jax.numpy as jnp
from jax import lax
from jax.experimental import pallas as pl
from jax.experimental.pallas import tpu as pltpu
from jax.experimental.pallas import tpu_sc as plsc

N_NODES = 50000
HALF = 25000
DIM = 64
PH = 25088
ROWS_PER_SUB = PH // 16
CHUNK = 128
N_SUB = 16
N_CORE = 2
EP = 802816
CHUNKS_PER_SUB = EP // (N_SUB * CHUNK)
DUMMY = HALF


def _layer_kernel(hpad, gidx, colloc, a_e, bpad, out, idx_v, cidx_v, a_v,
                  rows_v, shared, sem):
  cid = lax.axis_index("c")
  sid = lax.axis_index("s")

  my_rows = sid * ROWS_PER_SUB
  pltpu.sync_copy(bpad.at[pl.ds(cid * PH + my_rows, ROWS_PER_SUB)],
                  shared.at[pl.ds(my_rows, ROWS_PER_SUB)])
  plsc.subcore_barrier()

  def chunk_body(c, _):
    base = (sid * CHUNKS_PER_SUB + c) * CHUNK
    pltpu.sync_copy(gidx.at[pl.ds(base, CHUNK)], idx_v)
    pltpu.sync_copy(colloc.at[cid, pl.ds(base, CHUNK)], cidx_v)
    pltpu.sync_copy(a_e.at[pl.ds(base, CHUNK)], a_v)
    pltpu.async_copy(hpad.at[idx_v], rows_v, sem).wait()

    def group_body(g, _):
      av16 = a_v[pl.ds(g * 16, 16)]
      for j in range(16):
        e = g * 16 + j
        for d in range(DIM // 16):
          sl = pl.ds(d * 16, 16)
          rows_v[e, sl] = rows_v[e, sl] * av16[j]
      return 0

    lax.fori_loop(0, CHUNK // 16, group_body, 0)
    pltpu.sync_copy(rows_v, shared.at[cidx_v], add=True)
    return 0

  lax.fori_loop(0, CHUNKS_PER_SUB, chunk_body, 0)
  plsc.subcore_barrier()

  pltpu.sync_copy(shared.at[pl.ds(my_rows, ROWS_PER_SUB)],
                  out.at[pl.ds(cid * PH + my_rows, ROWS_PER_SUB)])


_layer = functools.partial(
    pl.kernel,
    out_type=jax.ShapeDtypeStruct((2 * PH, DIM), jnp.float32),
    mesh=plsc.VectorSubcoreMesh(core_axis_name="c", subcore_axis_name="s"),
    compiler_params=pltpu.CompilerParams(use_tc_tiling_on_sc=False),
    scratch_types=[
        pltpu.VMEM((CHUNK,), jnp.int32),
        pltpu.VMEM((CHUNK,), jnp.int32),
        pltpu.VMEM((CHUNK,), jnp.float32),
        pltpu.VMEM((CHUNK, DIM), jnp.float32),
        pltpu.VMEM_SHARED((PH, DIM), jnp.float32),
        pltpu.SemaphoreType.DMA,
    ],
)(_layer_kernel)


@jax.jit
def kernel(edge_index, edge_weight, edge_type, embeddings, edge_type_emb):
  row = edge_index[0].astype(jnp.int32)
  col = edge_index[1].astype(jnp.int32)
  n = embeddings.shape[0]
  x0 = embeddings

  deg = jnp.zeros((n,), jnp.float32).at[row].add(1.0)
  dsi = jnp.where(deg > 0, lax.rsqrt(jnp.where(deg > 0, deg, 1.0)), 0.0)
  a = edge_weight * dsi[row] * dsi[col]

  s = jnp.zeros((n, 3), jnp.float32).at[col, edge_type].add(a)
  b = 0.1 * (s @ edge_type_emb)

  gidx = jnp.where(row < HALF, row, row - HALF + PH)
  colloc = jnp.stack([
      jnp.where(col < HALF, col, DUMMY),
      jnp.where(col >= HALF, col - HALF, DUMMY),
  ])
  pad_e = EP - row.shape[0]
  gidx = jnp.concatenate([gidx, jnp.zeros((pad_e,), jnp.int32)])
  colloc = jnp.concatenate(
      [colloc, jnp.full((2, pad_e), DUMMY, jnp.int32)], axis=1)
  a_pad = jnp.concatenate([a, jnp.zeros((pad_e,), jnp.float32)])

  zpad = jnp.zeros((PH - HALF, DIM), jnp.float32)
  bpad = jnp.concatenate([b[:HALF], zpad, b[HALF:], zpad])
  hpad = jnp.concatenate([x0[:HALF], zpad, x0[HALF:], zpad])

  out = x0 / 4.0
  for _ in range(3):
    hpad = _layer(hpad, gidx, colloc, a_pad, bpad)
    out = out + jnp.concatenate([hpad[:HALF], hpad[PH:PH + HALF]]) / 4.0
  return out
